# Initial kernel scaffold; baseline (speedup 1.0000x reference)
import jax
import jax.numpy as jnp
from jax.experimental import pallas as pl
from jax.experimental.pallas import tpu as pltpu

NFEAT = 256
NHID = 32
NHEADS = 4
ALPHA = 0.2
N = 1024

BLK = 256
NB = N // BLK

_F32 = jnp.float32


def _neg_leaky(z):
    return jnp.where(z >= 0, -z, (-ALPHA) * z)


def _elu(x):
    return jnp.where(x > 0, x, jnp.expm1(x))


def _layer1_body(x_ref, ws_ref, a1_ref, a2_ref, adj_ref, out_ref,
                 h_ref, f1_ref, f2t_ref):
    i = pl.program_id(0)

    @pl.when(i == 0)
    def _init():
        for k in range(NHEADS):
            hk = jnp.dot(x_ref[...], ws_ref[k],
                         preferred_element_type=_F32)
            h_ref[:, k * NHID:(k + 1) * NHID] = hk
            f1_ref[:, k:k + 1] = jnp.dot(hk, a1_ref[:, k:k + 1],
                                         preferred_element_type=_F32)
            f2t_ref[k:k + 1, :] = jax.lax.dot_general(
                a2_ref[:, k:k + 1], hk,
                dimension_numbers=(((0,), (1,)), ((), ())),
                preferred_element_type=_F32)

    mask = adj_ref[...] != 0.0
    r0 = i * BLK
    for k in range(NHEADS):
        z = f1_ref[pl.ds(r0, BLK), k:k + 1] + f2t_ref[k:k + 1, :]
        e = jnp.where(mask, jnp.exp(_neg_leaky(z)), 0.0)
        s = jnp.sum(e, axis=1, keepdims=True)
        hk = h_ref[:, k * NHID:(k + 1) * NHID]
        hp = jnp.dot(e, hk, preferred_element_type=_F32) / s
        out_ref[:, k * NHID:(k + 1) * NHID] = _elu(hp)


def _layer2_body(h_ref, wout_ref, aout_ref, adj_ref, out_ref,
                 h2_ref, h2t_ref):
    i = pl.program_id(0)

    @pl.when(i == 0)
    def _init():
        h2_ref[...] = jnp.dot(h_ref[...], wout_ref[...],
                              preferred_element_type=_F32)
        h2t_ref[...] = jax.lax.dot_general(
            wout_ref[...], h_ref[...],
            dimension_numbers=(((0,), (1,)), ((), ())),
            preferred_element_type=_F32)

    a0 = aout_ref[0:1, 0:1]
    a1 = aout_ref[0:1, 1:2]
    mask = adj_ref[...] != 0.0
    r0 = i * BLK
    z = h2_ref[pl.ds(r0, BLK), :] * a0 + h2t_ref[...] * a1
    e = jnp.where(mask, jnp.exp(_neg_leaky(z)), 0.0)
    s = jnp.sum(e, axis=1, keepdims=True)
    num = jnp.sum(e * h2t_ref[...], axis=1, keepdims=True)
    out_ref[...] = jax.nn.sigmoid(_elu(num / s))


@jax.jit
def kernel(x, adj, Ws, attn_a, W_out, a_out):
    a1s = jnp.transpose(attn_a[:, 0, :NHID])
    a2s = jnp.transpose(attn_a[:, 0, NHID:])

    h = pl.pallas_call(
        _layer1_body,
        grid=(NB,),
        in_specs=[
            pl.BlockSpec((N, NFEAT), lambda i: (0, 0)),
            pl.BlockSpec((NHEADS, NFEAT, NHID), lambda i: (0, 0, 0)),
            pl.BlockSpec((NHID, NHEADS), lambda i: (0, 0)),
            pl.BlockSpec((NHID, NHEADS), lambda i: (0, 0)),
            pl.BlockSpec((BLK, N), lambda i: (i, 0)),
        ],
        out_specs=pl.BlockSpec((BLK, NHEADS * NHID), lambda i: (i, 0)),
        out_shape=jax.ShapeDtypeStruct((N, NHEADS * NHID), _F32),
        scratch_shapes=[
            pltpu.VMEM((N, NHEADS * NHID), _F32),
            pltpu.VMEM((N, NHEADS), _F32),
            pltpu.VMEM((NHEADS, N), _F32),
        ],
    )(x, Ws, a1s, a2s, adj)

    out = pl.pallas_call(
        _layer2_body,
        grid=(NB,),
        in_specs=[
            pl.BlockSpec((N, NHEADS * NHID), lambda i: (0, 0)),
            pl.BlockSpec((NHEADS * NHID, 1), lambda i: (0, 0)),
            pl.BlockSpec((1, 2), lambda i: (0, 0)),
            pl.BlockSpec((BLK, N), lambda i: (i, 0)),
        ],
        out_specs=pl.BlockSpec((BLK, 1), lambda i: (i, 0)),
        out_shape=jax.ShapeDtypeStruct((N, 1), _F32),
        scratch_shapes=[
            pltpu.VMEM((N, 1), _F32),
            pltpu.VMEM((1, N), _F32),
        ],
    )(h, W_out, a_out, adj)

    return out


# speedup vs baseline: 2379.8285x; 2379.8285x over previous
import jax
import jax.numpy as jnp
from jax.experimental import pallas as pl
from jax.experimental.pallas import tpu as pltpu

NFEAT = 256
NHID = 32
NHEADS = 4
ALPHA = 0.2
N = 1024

BLK = 256
NB = N // BLK

_F32 = jnp.float32


def _neg_leaky(z):
    return jnp.where(z >= 0, -z, (-ALPHA) * z)


def _elu(x):
    return jnp.where(x > 0, x, jnp.exp(jnp.minimum(x, 0.0)) - 1.0)


def _layer1_body(x_ref, ws_ref, a1_ref, a2_ref, adj_ref, out_ref,
                 h_ref, f1_ref, f2t_ref):
    i = pl.program_id(0)

    @pl.when(i == 0)
    def _init():
        for k in range(NHEADS):
            hk = jnp.dot(x_ref[...], ws_ref[k],
                         preferred_element_type=_F32)
            h_ref[:, k * NHID:(k + 1) * NHID] = hk
            f1_ref[:, k:k + 1] = jnp.dot(hk, a1_ref[:, k:k + 1],
                                         preferred_element_type=_F32)
            f2t_ref[k:k + 1, :] = jax.lax.dot_general(
                a2_ref[:, k:k + 1], hk,
                dimension_numbers=(((0,), (1,)), ((), ())),
                preferred_element_type=_F32)

    mask = adj_ref[...] != 0.0
    r0 = i * BLK
    for k in range(NHEADS):
        z = f1_ref[pl.ds(r0, BLK), k:k + 1] + f2t_ref[k:k + 1, :]
        e = jnp.where(mask, jnp.exp(_neg_leaky(z)), 0.0)
        s = jnp.sum(e, axis=1, keepdims=True)
        hk = h_ref[:, k * NHID:(k + 1) * NHID]
        hp = jnp.dot(e, hk, preferred_element_type=_F32) / s
        out_ref[:, k * NHID:(k + 1) * NHID] = _elu(hp)


def _layer2_body(h_ref, wout_ref, aout_ref, adj_ref, out_ref,
                 h2_ref, h2t_ref):
    i = pl.program_id(0)

    @pl.when(i == 0)
    def _init():
        h2_ref[...] = jnp.dot(h_ref[...], wout_ref[...],
                              preferred_element_type=_F32)
        h2t_ref[...] = jax.lax.dot_general(
            wout_ref[...], h_ref[...],
            dimension_numbers=(((0,), (1,)), ((), ())),
            preferred_element_type=_F32)

    a0 = aout_ref[0:1, 0:1]
    a1 = aout_ref[0:1, 1:2]
    mask = adj_ref[...] != 0.0
    r0 = i * BLK
    z = h2_ref[pl.ds(r0, BLK), :] * a0 + h2t_ref[...] * a1
    e = jnp.where(mask, jnp.exp(_neg_leaky(z)), 0.0)
    s = jnp.sum(e, axis=1, keepdims=True)
    num = jnp.sum(e * h2t_ref[...], axis=1, keepdims=True)
    out_ref[...] = jax.nn.sigmoid(_elu(num / s))


@jax.jit
def kernel(x, adj, Ws, attn_a, W_out, a_out):
    a1s = jnp.transpose(attn_a[:, 0, :NHID])
    a2s = jnp.transpose(attn_a[:, 0, NHID:])

    h = pl.pallas_call(
        _layer1_body,
        grid=(NB,),
        in_specs=[
            pl.BlockSpec((N, NFEAT), lambda i: (0, 0)),
            pl.BlockSpec((NHEADS, NFEAT, NHID), lambda i: (0, 0, 0)),
            pl.BlockSpec((NHID, NHEADS), lambda i: (0, 0)),
            pl.BlockSpec((NHID, NHEADS), lambda i: (0, 0)),
            pl.BlockSpec((BLK, N), lambda i: (i, 0)),
        ],
        out_specs=pl.BlockSpec((BLK, NHEADS * NHID), lambda i: (i, 0)),
        out_shape=jax.ShapeDtypeStruct((N, NHEADS * NHID), _F32),
        scratch_shapes=[
            pltpu.VMEM((N, NHEADS * NHID), _F32),
            pltpu.VMEM((N, NHEADS), _F32),
            pltpu.VMEM((NHEADS, N), _F32),
        ],
    )(x, Ws, a1s, a2s, adj)

    out = pl.pallas_call(
        _layer2_body,
        grid=(NB,),
        in_specs=[
            pl.BlockSpec((N, NHEADS * NHID), lambda i: (0, 0)),
            pl.BlockSpec((NHEADS * NHID, 1), lambda i: (0, 0)),
            pl.BlockSpec((1, 2), lambda i: (0, 0)),
            pl.BlockSpec((BLK, N), lambda i: (i, 0)),
        ],
        out_specs=pl.BlockSpec((BLK, 1), lambda i: (i, 0)),
        out_shape=jax.ShapeDtypeStruct((N, 1), _F32),
        scratch_shapes=[
            pltpu.VMEM((N, 1), _F32),
            pltpu.VMEM((1, N), _F32),
        ],
    )(h, W_out, a_out, adj)

    return out


# speedup vs baseline: 2923.7283x; 1.2285x over previous
import jax
import jax.numpy as jnp
from jax.experimental import pallas as pl
from jax.experimental.pallas import tpu as pltpu

NFEAT = 256
NHID = 32
NHEADS = 4
ALPHA = 0.2
N = 1024

BLK = 256
NB = N // BLK
SLOT = 128

_F32 = jnp.float32


def _elu(x):
    return jnp.where(x > 0, x, jnp.exp(jnp.minimum(x, 0.0)) - 1.0)


def _gat_body(x_ref, ws_ref, a1_ref, a2_ref, adj_ref, wout_ref, aout_ref,
              out_ref, haug_ref, f1n_ref, f2tn_ref, hl2_ref,
              h2a_ref, h2tb_ref, haug2_ref):
    i = pl.program_id(0)

    @pl.when(i == 0)
    def _init1():
        ones_col = jnp.ones((N, 1), dtype=_F32)
        for k in range(NHEADS):
            hk = jnp.dot(x_ref[...], ws_ref[k],
                         preferred_element_type=_F32)
            haug_ref[:, k * SLOT:k * SLOT + NHID] = hk
            haug_ref[:, k * SLOT + NHID:k * SLOT + NHID + 1] = ones_col
            f1n_ref[:, k:k + 1] = jnp.dot(hk, -a1_ref[:, k:k + 1],
                                          preferred_element_type=_F32)
            f2tn_ref[k:k + 1, :] = jax.lax.dot_general(
                -a2_ref[:, k:k + 1], hk,
                dimension_numbers=(((0,), (1,)), ((), ())),
                preferred_element_type=_F32)

    @pl.when(i == NB)
    def _init2():
        h2 = jnp.dot(hl2_ref[...], wout_ref[...],
                     preferred_element_type=_F32)
        h2t = jax.lax.dot_general(
            wout_ref[...], hl2_ref[...],
            dimension_numbers=(((0,), (1,)), ((), ())),
            preferred_element_type=_F32)
        h2a_ref[...] = h2 * (-aout_ref[0:1, 0:1])
        h2tb_ref[...] = h2t * (-aout_ref[0:1, 1:2])
        haug2_ref[:, 0:1] = h2
        haug2_ref[:, 1:2] = jnp.ones((N, 1), dtype=_F32)

    ib = jnp.where(i < NB, i, i - NB)
    r0 = ib * BLK
    adj_blk = adj_ref[pl.ds(r0, BLK), :]

    @pl.when(i < NB)
    def _layer1():
        for k in range(NHEADS):
            nz = f1n_ref[pl.ds(r0, BLK), k:k + 1] + f2tn_ref[k:k + 1, :]
            e = jnp.exp(jnp.minimum(nz, ALPHA * nz)) * adj_blk
            ns = jnp.dot(e, haug_ref[:, k * SLOT:(k + 1) * SLOT],
                         preferred_element_type=_F32)
            hp = ns[:, :NHID] / ns[:, NHID:NHID + 1]
            hl2_ref[pl.ds(r0, BLK), k * NHID:(k + 1) * NHID] = _elu(hp)

    @pl.when(i >= NB)
    def _layer2():
        nz = h2a_ref[pl.ds(r0, BLK), :] + h2tb_ref[...]
        e = jnp.exp(jnp.minimum(nz, ALPHA * nz)) * adj_blk
        ns = jnp.dot(e, haug2_ref[...],
                     preferred_element_type=_F32)
        out_ref[...] = jax.nn.sigmoid(_elu(ns[:, 0:1] / ns[:, 1:2]))


@jax.jit
def kernel(x, adj, Ws, attn_a, W_out, a_out):
    a1s = jnp.transpose(attn_a[:, 0, :NHID])
    a2s = jnp.transpose(attn_a[:, 0, NHID:])

    out = pl.pallas_call(
        _gat_body,
        grid=(2 * NB,),
        in_specs=[
            pl.BlockSpec((N, NFEAT), lambda i: (0, 0)),
            pl.BlockSpec((NHEADS, NFEAT, NHID), lambda i: (0, 0, 0)),
            pl.BlockSpec((NHID, NHEADS), lambda i: (0, 0)),
            pl.BlockSpec((NHID, NHEADS), lambda i: (0, 0)),
            pl.BlockSpec((N, N), lambda i: (0, 0)),
            pl.BlockSpec((NHEADS * NHID, 1), lambda i: (0, 0)),
            pl.BlockSpec((1, 2), lambda i: (0, 0)),
        ],
        out_specs=pl.BlockSpec((BLK, 1), lambda i: (jnp.maximum(i - NB, 0), 0)),
        out_shape=jax.ShapeDtypeStruct((N, 1), _F32),
        scratch_shapes=[
            pltpu.VMEM((N, NHEADS * SLOT), _F32),
            pltpu.VMEM((N, NHEADS), _F32),
            pltpu.VMEM((NHEADS, N), _F32),
            pltpu.VMEM((N, NHEADS * NHID), _F32),
            pltpu.VMEM((N, 1), _F32),
            pltpu.VMEM((1, N), _F32),
            pltpu.VMEM((N, SLOT), _F32),
        ],
    )(x, Ws, a1s, a2s, adj, W_out, a_out)

    return out


# speedup vs baseline: 3139.7868x; 1.0739x over previous
import jax
import jax.numpy as jnp
from jax.experimental import pallas as pl
from jax.experimental.pallas import tpu as pltpu

NFEAT = 256
NHID = 32
NHEADS = 4
ALPHA = 0.2
N = 1024

BLK = 512
NB = N // BLK
SLOT = 128

_F32 = jnp.float32


def _elu(x):
    return jnp.where(x > 0, x, jnp.exp(jnp.minimum(x, 0.0)) - 1.0)


def _gat_body(x_ref, ws_ref, a1_ref, a2_ref, adj_ref, wout_ref, aout_ref,
              out_ref, haug_ref, f1n_ref, f2tn_ref, hl2_ref,
              h2a_ref, h2tb_ref, haug2_ref):
    i = pl.program_id(0)

    @pl.when(i == 0)
    def _init1():
        ones_col = jnp.ones((N, 1), dtype=_F32)
        for k in range(NHEADS):
            hk = jnp.dot(x_ref[...], ws_ref[k],
                         preferred_element_type=_F32)
            haug_ref[:, k * SLOT:k * SLOT + NHID] = hk
            haug_ref[:, k * SLOT + NHID:k * SLOT + NHID + 1] = ones_col
            f1n_ref[:, k:k + 1] = jnp.dot(hk, -a1_ref[:, k:k + 1],
                                          preferred_element_type=_F32)
            f2tn_ref[k:k + 1, :] = jax.lax.dot_general(
                -a2_ref[:, k:k + 1], hk,
                dimension_numbers=(((0,), (1,)), ((), ())),
                preferred_element_type=_F32)

    @pl.when(i == NB)
    def _init2():
        h2 = jnp.dot(hl2_ref[...], wout_ref[...],
                     preferred_element_type=_F32)
        h2t = jax.lax.dot_general(
            wout_ref[...], hl2_ref[...],
            dimension_numbers=(((0,), (1,)), ((), ())),
            preferred_element_type=_F32)
        h2a_ref[...] = h2 * (-aout_ref[0:1, 0:1])
        h2tb_ref[...] = h2t * (-aout_ref[0:1, 1:2])
        haug2_ref[:, 0:1] = h2
        haug2_ref[:, 1:2] = jnp.ones((N, 1), dtype=_F32)

    ib = jnp.where(i < NB, i, i - NB)
    r0 = ib * BLK
    adj_blk = adj_ref[pl.ds(r0, BLK), :]

    @pl.when(i < NB)
    def _layer1():
        for k in range(NHEADS):
            nz = f1n_ref[pl.ds(r0, BLK), k:k + 1] + f2tn_ref[k:k + 1, :]
            e = jnp.exp(jnp.minimum(nz, ALPHA * nz)) * adj_blk
            ns = jnp.dot(e, haug_ref[:, k * SLOT:(k + 1) * SLOT],
                         preferred_element_type=_F32)
            hp = ns[:, :NHID] / ns[:, NHID:NHID + 1]
            hl2_ref[pl.ds(r0, BLK), k * NHID:(k + 1) * NHID] = _elu(hp)

    @pl.when(i >= NB)
    def _layer2():
        nz = h2a_ref[pl.ds(r0, BLK), :] + h2tb_ref[...]
        e = jnp.exp(jnp.minimum(nz, ALPHA * nz)) * adj_blk
        ns = jnp.dot(e, haug2_ref[...],
                     preferred_element_type=_F32)
        out_ref[...] = jax.nn.sigmoid(_elu(ns[:, 0:1] / ns[:, 1:2]))


@jax.jit
def kernel(x, adj, Ws, attn_a, W_out, a_out):
    a1s = jnp.transpose(attn_a[:, 0, :NHID])
    a2s = jnp.transpose(attn_a[:, 0, NHID:])

    out = pl.pallas_call(
        _gat_body,
        grid=(2 * NB,),
        in_specs=[
            pl.BlockSpec((N, NFEAT), lambda i: (0, 0)),
            pl.BlockSpec((NHEADS, NFEAT, NHID), lambda i: (0, 0, 0)),
            pl.BlockSpec((NHID, NHEADS), lambda i: (0, 0)),
            pl.BlockSpec((NHID, NHEADS), lambda i: (0, 0)),
            pl.BlockSpec((N, N), lambda i: (0, 0)),
            pl.BlockSpec((NHEADS * NHID, 1), lambda i: (0, 0)),
            pl.BlockSpec((1, 2), lambda i: (0, 0)),
        ],
        out_specs=pl.BlockSpec((BLK, 1), lambda i: (jnp.maximum(i - NB, 0), 0)),
        out_shape=jax.ShapeDtypeStruct((N, 1), _F32),
        scratch_shapes=[
            pltpu.VMEM((N, NHEADS * SLOT), _F32),
            pltpu.VMEM((N, NHEADS), _F32),
            pltpu.VMEM((NHEADS, N), _F32),
            pltpu.VMEM((N, NHEADS * NHID), _F32),
            pltpu.VMEM((N, 1), _F32),
            pltpu.VMEM((1, N), _F32),
            pltpu.VMEM((N, SLOT), _F32),
        ],
    )(x, Ws, a1s, a2s, adj, W_out, a_out)

    return out


# speedup vs baseline: 3203.2557x; 1.0202x over previous
import jax
import jax.numpy as jnp
from jax.experimental import pallas as pl
from jax.experimental.pallas import tpu as pltpu

NFEAT = 256
NHID = 32
NHEADS = 4
ALPHA = 0.2
N = 1024

BLK = 1024
NB = N // BLK
SLOT = 128

_F32 = jnp.float32


def _elu(x):
    return jnp.where(x > 0, x, jnp.exp(jnp.minimum(x, 0.0)) - 1.0)


def _gat_body(x_ref, ws_ref, a1_ref, a2_ref, adj_ref, wout_ref, aout_ref,
              out_ref, haug_ref, f1n_ref, f2tn_ref, hl2_ref,
              h2a_ref, h2tb_ref, haug2_ref):
    i = pl.program_id(0)

    @pl.when(i == 0)
    def _init1():
        ones_col = jnp.ones((N, 1), dtype=_F32)
        for k in range(NHEADS):
            hk = jnp.dot(x_ref[...], ws_ref[k],
                         preferred_element_type=_F32)
            haug_ref[:, k * SLOT:k * SLOT + NHID] = hk
            haug_ref[:, k * SLOT + NHID:k * SLOT + NHID + 1] = ones_col
            f1n_ref[:, k:k + 1] = jnp.dot(hk, -a1_ref[:, k:k + 1],
                                          preferred_element_type=_F32)
            f2tn_ref[k:k + 1, :] = jax.lax.dot_general(
                -a2_ref[:, k:k + 1], hk,
                dimension_numbers=(((0,), (1,)), ((), ())),
                preferred_element_type=_F32)

    @pl.when(i == NB)
    def _init2():
        h2 = jnp.dot(hl2_ref[...], wout_ref[...],
                     preferred_element_type=_F32)
        h2t = jax.lax.dot_general(
            wout_ref[...], hl2_ref[...],
            dimension_numbers=(((0,), (1,)), ((), ())),
            preferred_element_type=_F32)
        h2a_ref[...] = h2 * (-aout_ref[0:1, 0:1])
        h2tb_ref[...] = h2t * (-aout_ref[0:1, 1:2])
        haug2_ref[:, 0:1] = h2
        haug2_ref[:, 1:2] = jnp.ones((N, 1), dtype=_F32)

    ib = jnp.where(i < NB, i, i - NB)
    r0 = ib * BLK
    adj_blk = adj_ref[pl.ds(r0, BLK), :]

    @pl.when(i < NB)
    def _layer1():
        for k in range(NHEADS):
            nz = f1n_ref[pl.ds(r0, BLK), k:k + 1] + f2tn_ref[k:k + 1, :]
            e = jnp.exp(jnp.minimum(nz, ALPHA * nz)) * adj_blk
            ns = jnp.dot(e, haug_ref[:, k * SLOT:(k + 1) * SLOT],
                         preferred_element_type=_F32)
            hp = ns[:, :NHID] / ns[:, NHID:NHID + 1]
            hl2_ref[pl.ds(r0, BLK), k * NHID:(k + 1) * NHID] = _elu(hp)

    @pl.when(i >= NB)
    def _layer2():
        nz = h2a_ref[pl.ds(r0, BLK), :] + h2tb_ref[...]
        e = jnp.exp(jnp.minimum(nz, ALPHA * nz)) * adj_blk
        ns = jnp.dot(e, haug2_ref[...],
                     preferred_element_type=_F32)
        out_ref[...] = jax.nn.sigmoid(_elu(ns[:, 0:1] / ns[:, 1:2]))


@jax.jit
def kernel(x, adj, Ws, attn_a, W_out, a_out):
    a1s = jnp.transpose(attn_a[:, 0, :NHID])
    a2s = jnp.transpose(attn_a[:, 0, NHID:])

    out = pl.pallas_call(
        _gat_body,
        grid=(2 * NB,),
        in_specs=[
            pl.BlockSpec((N, NFEAT), lambda i: (0, 0)),
            pl.BlockSpec((NHEADS, NFEAT, NHID), lambda i: (0, 0, 0)),
            pl.BlockSpec((NHID, NHEADS), lambda i: (0, 0)),
            pl.BlockSpec((NHID, NHEADS), lambda i: (0, 0)),
            pl.BlockSpec((N, N), lambda i: (0, 0)),
            pl.BlockSpec((NHEADS * NHID, 1), lambda i: (0, 0)),
            pl.BlockSpec((1, 2), lambda i: (0, 0)),
        ],
        out_specs=pl.BlockSpec((BLK, 1), lambda i: (jnp.maximum(i - NB, 0), 0)),
        out_shape=jax.ShapeDtypeStruct((N, 1), _F32),
        scratch_shapes=[
            pltpu.VMEM((N, NHEADS * SLOT), _F32),
            pltpu.VMEM((N, NHEADS), _F32),
            pltpu.VMEM((NHEADS, N), _F32),
            pltpu.VMEM((N, NHEADS * NHID), _F32),
            pltpu.VMEM((N, 1), _F32),
            pltpu.VMEM((1, N), _F32),
            pltpu.VMEM((N, SLOT), _F32),
        ],
    )(x, Ws, a1s, a2s, adj, W_out, a_out)

    return out


# speedup vs baseline: 3271.1959x; 1.0212x over previous
import jax
import jax.numpy as jnp
from jax.experimental import pallas as pl
from jax.experimental.pallas import tpu as pltpu

NFEAT = 256
NHID = 32
NHEADS = 4
ALPHA = 0.2
N = 1024

BLK = 1024
NB = N // BLK
SLOT = 128
CLIP = 60.0

_F32 = jnp.float32


def _elu(x):
    return jnp.where(x > 0, x, jnp.exp(jnp.minimum(x, 0.0)) - 1.0)


def _cexp(z):
    return jnp.exp(jnp.clip(z, -CLIP, CLIP))


def _gat_body(x_ref, ws_ref, a1_ref, a2_ref, adj_ref, wout_ref, aout_ref,
              out_ref, haug_ref, e1_ref, e1a_ref, e2t_ref, e2ta_ref,
              hl2_ref, g1_ref, g1a_ref, g2t_ref, g2ta_ref, haug2_ref):
    i = pl.program_id(0)

    @pl.when(i == 0)
    def _init1():
        ones_col = jnp.ones((N, 1), dtype=_F32)
        for k in range(NHEADS):
            hk = jnp.dot(x_ref[...], ws_ref[k],
                         preferred_element_type=_F32)
            haug_ref[:, k * SLOT:k * SLOT + NHID] = hk
            haug_ref[:, k * SLOT + NHID:k * SLOT + NHID + 1] = ones_col
            f1n = jnp.dot(hk, -a1_ref[:, k:k + 1],
                          preferred_element_type=_F32)
            f2tn = jax.lax.dot_general(
                -a2_ref[:, k:k + 1], hk,
                dimension_numbers=(((0,), (1,)), ((), ())),
                preferred_element_type=_F32)
            e1_ref[:, k:k + 1] = _cexp(f1n)
            e1a_ref[:, k:k + 1] = _cexp(ALPHA * f1n)
            e2t_ref[k:k + 1, :] = _cexp(f2tn)
            e2ta_ref[k:k + 1, :] = _cexp(ALPHA * f2tn)

    @pl.when(i == NB)
    def _init2():
        h2 = jnp.dot(hl2_ref[...], wout_ref[...],
                     preferred_element_type=_F32)
        h2t = jax.lax.dot_general(
            wout_ref[...], hl2_ref[...],
            dimension_numbers=(((0,), (1,)), ((), ())),
            preferred_element_type=_F32)
        z1 = h2 * (-aout_ref[0:1, 0:1])
        z2 = h2t * (-aout_ref[0:1, 1:2])
        g1_ref[...] = _cexp(z1)
        g1a_ref[...] = _cexp(ALPHA * z1)
        g2t_ref[...] = _cexp(z2)
        g2ta_ref[...] = _cexp(ALPHA * z2)
        haug2_ref[:, 0:1] = h2
        haug2_ref[:, 1:2] = jnp.ones((N, 1), dtype=_F32)

    ib = jnp.where(i < NB, i, i - NB)
    r0 = ib * BLK
    adj_blk = adj_ref[pl.ds(r0, BLK), :]

    @pl.when(i < NB)
    def _layer1():
        for k in range(NHEADS):
            p1 = e1_ref[pl.ds(r0, BLK), k:k + 1] * e2t_ref[k:k + 1, :]
            p2 = e1a_ref[pl.ds(r0, BLK), k:k + 1] * e2ta_ref[k:k + 1, :]
            e = jnp.minimum(p1, p2) * adj_blk
            ns = jnp.dot(e, haug_ref[:, k * SLOT:(k + 1) * SLOT],
                         preferred_element_type=_F32)
            hp = ns[:, :NHID] / ns[:, NHID:NHID + 1]
            hl2_ref[pl.ds(r0, BLK), k * NHID:(k + 1) * NHID] = _elu(hp)

    @pl.when(i >= NB)
    def _layer2():
        p1 = g1_ref[pl.ds(r0, BLK), :] * g2t_ref[...]
        p2 = g1a_ref[pl.ds(r0, BLK), :] * g2ta_ref[...]
        e = jnp.minimum(p1, p2) * adj_blk
        ns = jnp.dot(e, haug2_ref[...],
                     preferred_element_type=_F32)
        out_ref[...] = jax.nn.sigmoid(_elu(ns[:, 0:1] / ns[:, 1:2]))


@jax.jit
def kernel(x, adj, Ws, attn_a, W_out, a_out):
    a1s = jnp.transpose(attn_a[:, 0, :NHID])
    a2s = jnp.transpose(attn_a[:, 0, NHID:])

    out = pl.pallas_call(
        _gat_body,
        grid=(2 * NB,),
        in_specs=[
            pl.BlockSpec((N, NFEAT), lambda i: (0, 0)),
            pl.BlockSpec((NHEADS, NFEAT, NHID), lambda i: (0, 0, 0)),
            pl.BlockSpec((NHID, NHEADS), lambda i: (0, 0)),
            pl.BlockSpec((NHID, NHEADS), lambda i: (0, 0)),
            pl.BlockSpec((N, N), lambda i: (0, 0)),
            pl.BlockSpec((NHEADS * NHID, 1), lambda i: (0, 0)),
            pl.BlockSpec((1, 2), lambda i: (0, 0)),
        ],
        out_specs=pl.BlockSpec((BLK, 1), lambda i: (jnp.maximum(i - NB, 0), 0)),
        out_shape=jax.ShapeDtypeStruct((N, 1), _F32),
        scratch_shapes=[
            pltpu.VMEM((N, NHEADS * SLOT), _F32),
            pltpu.VMEM((N, NHEADS), _F32),
            pltpu.VMEM((N, NHEADS), _F32),
            pltpu.VMEM((NHEADS, N), _F32),
            pltpu.VMEM((NHEADS, N), _F32),
            pltpu.VMEM((N, NHEADS * NHID), _F32),
            pltpu.VMEM((N, 1), _F32),
            pltpu.VMEM((N, 1), _F32),
            pltpu.VMEM((1, N), _F32),
            pltpu.VMEM((1, N), _F32),
            pltpu.VMEM((N, SLOT), _F32),
        ],
    )(x, Ws, a1s, a2s, adj, W_out, a_out)

    return out
